# baseline (speedup 1.0000x reference)
import jax
import jax.numpy as jnp
from jax import lax
from jax.experimental import pallas as pl
from jax.experimental.pallas import tpu as pltpu
from jax.experimental.pallas import tpu_sc as plsc

NUM_POSES = 100000
BATCH = 4096

_INFO = plsc.get_sparse_core_info()
_NC = _INFO.num_cores
_NS = _INFO.num_subcores
_NW = _NC * _NS
_BPW = BATCH // _NW
_L = 16


def _gather_body(idx_hbm, flat_hbm, qo_hbm, to_hbm,
                 iv, qcols, tcols, sem):
    wid = lax.axis_index("s") * _NC + lax.axis_index("c")
    base = wid * _BPW
    pltpu.sync_copy(idx_hbm.at[pl.ds(base, _BPW)], iv)

    qcps = [pltpu.async_copy(
        flat_hbm.at[pl.ds(c * NUM_POSES, NUM_POSES)].at[iv],
        qcols.at[pl.ds(c * _BPW, _BPW)], sem) for c in range(4)]
    tcps = [pltpu.async_copy(
        flat_hbm.at[pl.ds((4 + c) * NUM_POSES, NUM_POSES)].at[iv],
        tcols.at[pl.ds(c * _BPW, _BPW)], sem) for c in range(3)]
    for cp in qcps:
        cp.wait()
    pltpu.sync_copy(qcols, qo_hbm.at[pl.ds(wid * (_BPW * 4), _BPW * 4)])
    for cp in tcps:
        cp.wait()
    pltpu.sync_copy(tcols, to_hbm.at[pl.ds(wid * (_BPW * 4), _BPW * 3)])


@jax.jit
def kernel(camera_pose_indices, q_pointcloud_camera_table, t_pointcloud_camera_table):
    idx = camera_pose_indices.astype(jnp.int32)
    gather = pl.kernel(
        _gather_body,
        out_type=(
            jax.ShapeDtypeStruct((4 * BATCH,), jnp.float32),
            jax.ShapeDtypeStruct((4 * BATCH,), jnp.float32),
        ),
        mesh=plsc.VectorSubcoreMesh(core_axis_name="c", subcore_axis_name="s"),
        scratch_types=[
            pltpu.VMEM((_BPW,), jnp.int32),
            pltpu.VMEM((_BPW * 4,), jnp.float32),
            pltpu.VMEM((_BPW * 3,), jnp.float32),
            pltpu.SemaphoreType.DMA,
        ],
        compiler_params=pltpu.CompilerParams(needs_layout_passes=False,
                                             skip_device_barrier=True),
    )
    flat = jnp.concatenate([q_pointcloud_camera_table.T,
                            t_pointcloud_camera_table.T], axis=0).reshape(-1)
    qo, to = gather(idx, flat)
    q = jnp.transpose(qo.reshape(_NW, 4, _BPW), (0, 2, 1)).reshape(BATCH, 4)
    t = jnp.transpose(to.reshape(_NW, 4, _BPW), (0, 2, 1)).reshape(BATCH, 4)[:, :3]
    return q, t
